# Initial kernel scaffold; baseline (speedup 1.0000x reference)
import jax
import jax.numpy as jnp
from jax.experimental import pallas as pl


def kernel(features, cut_zs, cut_ys, cut_xs, invalid_mask):
    raise NotImplementedError("write your pallas kernel here")


# speedup vs baseline: 9.5670x; 9.5670x over previous
import functools

import jax
import jax.numpy as jnp
from jax import lax
from jax.experimental import pallas as pl
from jax.experimental.pallas import tpu as pltpu
from jax.experimental.pallas import tpu_sc as plsc

B = 16
C = 64
H = 11
W = 20
P = 2784
HW = H * W
L = C * H
NC = 2
NS = 16
NW = NC * NS
PT = P // NW
LANES = 16
SROW = LANES
FSIZE = C * HW
FPAD = FSIZE + LANES
SENTINEL = 1 << 20


def _sc_body(feats_hbm, scode_hbm, out_hbm, scode_v, fbuf, obuf):
    cid = lax.axis_index("c")
    sid = lax.axis_index("s")
    wid = sid * NC + cid
    p0 = wid * PT

    pltpu.sync_copy(scode_hbm.at[pl.ds(p0 * SROW, PT * SROW)], scode_v)
    fbuf[pl.ds(FSIZE, LANES)] = jnp.zeros((LANES,), jnp.float32)

    lane = lax.iota(jnp.int32, LANES)
    gidx_c = [(lane + kk * LANES) * HW for kk in range(4)]
    sidx_c = [[(lane + kk * LANES) * H + h for kk in range(4)]
              for h in range(H)]
    limit = jnp.full((LANES,), FSIZE, jnp.int32)

    def row_body(r, _):
        rs = r * SROW
        ro = r * L
        for h in range(H):
            svec = plsc.load_gather(scode_v, [jnp.full((LANES,), rs + h,
                                                       jnp.int32)])
            for kk in range(4):
                idx = jnp.minimum(gidx_c[kk] + svec, limit)
                vals = plsc.load_gather(fbuf, [idx])
                plsc.store_scatter(obuf, [sidx_c[h][kk] + ro], vals)
        return 0

    def batch_body(b, _):
        pltpu.sync_copy(feats_hbm.at[pl.ds(b * FSIZE, FSIZE)],
                        fbuf.at[pl.ds(0, FSIZE)])
        lax.fori_loop(0, PT, row_body, 0)
        pltpu.sync_copy(obuf, out_hbm.at[pl.ds((b * P + p0) * L, PT * L)])
        return 0

    lax.fori_loop(0, B, batch_body, 0)


_sc_gather = functools.partial(
    pl.kernel,
    out_type=jax.ShapeDtypeStruct((B * P * L,), jnp.float32),
    mesh=plsc.VectorSubcoreMesh(core_axis_name="c", subcore_axis_name="s",
                                num_cores=NC, num_subcores=NS),
    compiler_params=pltpu.CompilerParams(needs_layout_passes=False),
    scratch_types=[
        pltpu.VMEM((PT * SROW,), jnp.int32),
        pltpu.VMEM((FPAD,), jnp.float32),
        pltpu.VMEM((PT * L,), jnp.float32),
    ],
)(_sc_body)


def kernel(features, cut_zs, cut_ys, cut_xs, invalid_mask):
    del cut_zs, cut_ys
    xs = cut_xs.reshape(P, C, H)[:, 0, :]
    inv = invalid_mask.reshape(P, C, H)[:, 0, :]
    h_off = (jnp.arange(H, dtype=jnp.int32) * W)[None, :]
    scode = jnp.where(inv, jnp.int32(SENTINEL), xs + h_off)
    scode = jnp.pad(scode, ((0, 0), (0, SROW - H))).reshape(-1)
    feats_flat = features.reshape(-1)
    out = _sc_gather(feats_flat, scode)
    return out.reshape(B, P, C, H, 1)


# speedup vs baseline: 10.1489x; 1.0608x over previous
import functools

import jax
import jax.numpy as jnp
from jax import lax
from jax.experimental import pallas as pl
from jax.experimental.pallas import tpu as pltpu
from jax.experimental.pallas import tpu_sc as plsc

B = 16
C = 64
H = 11
W = 20
P = 2784
HW = H * W
L = C * H
NC = 2
NS = 16
NW = NC * NS
PT = P // NW
LANES = 16
SROW = LANES
FSIZE = C * HW
FPAD = FSIZE + LANES
SENTINEL = 1 << 20


def _sc_body(feats_hbm, scode_hbm, out_hbm, scode_v, fbuf, obuf):
    cid = lax.axis_index("c")
    sid = lax.axis_index("s")
    wid = sid * NC + cid
    p0 = wid * PT

    pltpu.sync_copy(scode_hbm.at[pl.ds(p0 * SROW, PT * SROW)], scode_v)
    fbuf[pl.ds(FSIZE, LANES)] = jnp.zeros((LANES,), jnp.float32)

    lane = lax.iota(jnp.int32, LANES)
    gidx_c = [(lane + kk * LANES) * HW for kk in range(4)]
    sidx_c = [[(lane + kk * LANES) * H + h for kk in range(4)]
              for h in range(H)]
    limit = jnp.full((LANES,), FSIZE, jnp.int32)

    def batch_body(b, _):
        pltpu.sync_copy(feats_hbm.at[pl.ds(b * FSIZE, FSIZE)],
                        fbuf.at[pl.ds(0, FSIZE)])

        @plsc.parallel_loop(0, PT, 1, unroll=4)
        def row_body(r):
            rs = r * SROW
            ro = r * L
            for h in range(H):
                svec = plsc.load_gather(scode_v, [jnp.full((LANES,), rs + h,
                                                           jnp.int32)])
                for kk in range(4):
                    idx = jnp.minimum(gidx_c[kk] + svec, limit)
                    vals = plsc.load_gather(fbuf, [idx])
                    plsc.store_scatter(obuf, [sidx_c[h][kk] + ro], vals)

        pltpu.sync_copy(obuf, out_hbm.at[pl.ds((b * P + p0) * L, PT * L)])
        return 0

    lax.fori_loop(0, B, batch_body, 0)


_sc_gather = functools.partial(
    pl.kernel,
    out_type=jax.ShapeDtypeStruct((B * P * L,), jnp.float32),
    mesh=plsc.VectorSubcoreMesh(core_axis_name="c", subcore_axis_name="s",
                                num_cores=NC, num_subcores=NS),
    compiler_params=pltpu.CompilerParams(needs_layout_passes=False),
    scratch_types=[
        pltpu.VMEM((PT * SROW,), jnp.int32),
        pltpu.VMEM((FPAD,), jnp.float32),
        pltpu.VMEM((PT * L,), jnp.float32),
    ],
)(_sc_body)


def kernel(features, cut_zs, cut_ys, cut_xs, invalid_mask):
    del cut_zs, cut_ys
    xs = cut_xs.reshape(P, C, H)[:, 0, :]
    inv = invalid_mask.reshape(P, C, H)[:, 0, :]
    h_off = (jnp.arange(H, dtype=jnp.int32) * W)[None, :]
    scode = jnp.where(inv, jnp.int32(SENTINEL), xs + h_off)
    scode = jnp.pad(scode, ((0, 0), (0, SROW - H))).reshape(-1)
    feats_flat = features.reshape(-1)
    out = _sc_gather(feats_flat, scode)
    return out.reshape(B, P, C, H, 1)


# speedup vs baseline: 10.1494x; 1.0001x over previous
import functools

import jax
import jax.numpy as jnp
from jax import lax
from jax.experimental import pallas as pl
from jax.experimental.pallas import tpu as pltpu
from jax.experimental.pallas import tpu_sc as plsc

B = 16
C = 64
H = 11
W = 20
P = 2784
HW = H * W
L = C * H
NC = 2
NS = 16
NW = NC * NS
PT = P // NW
LANES = 16
SROW = LANES
FSIZE = C * HW
FPAD = FSIZE + LANES
SENTINEL = 1 << 20


def _sc_body(feats_hbm, scode_hbm, out_hbm, scode_v, fbuf, obuf):
    cid = lax.axis_index("c")
    sid = lax.axis_index("s")
    wid = sid * NC + cid
    p0 = wid * PT

    pltpu.sync_copy(scode_hbm.at[pl.ds(p0 * SROW, PT * SROW)], scode_v)
    fbuf[pl.ds(FSIZE, LANES)] = jnp.zeros((LANES,), jnp.float32)

    lane = lax.iota(jnp.int32, LANES)
    gidx_c = [(lane + kk * LANES) * HW for kk in range(4)]
    sidx_c = [[(lane + kk * LANES) * H + h for kk in range(4)]
              for h in range(H)]
    limit = jnp.full((LANES,), FSIZE, jnp.int32)

    def batch_body(b, _):
        pltpu.sync_copy(feats_hbm.at[pl.ds(b * FSIZE, FSIZE)],
                        fbuf.at[pl.ds(0, FSIZE)])

        @plsc.parallel_loop(0, PT, 1, unroll=4)
        def row_body(r):
            rs = r * SROW
            ro = r * L
            for h in range(H):
                svec = plsc.load_gather(scode_v, [jnp.full((LANES,), rs + h,
                                                           jnp.int32)])
                for kk in range(4):
                    idx = jnp.minimum(gidx_c[kk] + svec, limit)
                    vals = plsc.load_gather(fbuf, [idx])
                    plsc.store_scatter(obuf, [sidx_c[h][kk] + ro], vals)

        pltpu.sync_copy(obuf, out_hbm.at[pl.ds((b * P + p0) * L, PT * L)])
        return 0

    lax.fori_loop(0, B, batch_body, 0)


_sc_gather = functools.partial(
    pl.kernel,
    out_type=jax.ShapeDtypeStruct((B * P * L,), jnp.float32),
    mesh=plsc.VectorSubcoreMesh(core_axis_name="c", subcore_axis_name="s",
                                num_cores=NC, num_subcores=NS),
    compiler_params=pltpu.CompilerParams(needs_layout_passes=False,
                                         disable_bounds_checks=True),
    scratch_types=[
        pltpu.VMEM((PT * SROW,), jnp.int32),
        pltpu.VMEM((FPAD,), jnp.float32),
        pltpu.VMEM((PT * L,), jnp.float32),
    ],
)(_sc_body)


def kernel(features, cut_zs, cut_ys, cut_xs, invalid_mask):
    del cut_zs, cut_ys
    xs = cut_xs.reshape(P, C, H)[:, 0, :]
    inv = invalid_mask.reshape(P, C, H)[:, 0, :]
    h_off = (jnp.arange(H, dtype=jnp.int32) * W)[None, :]
    scode = jnp.where(inv, jnp.int32(SENTINEL), xs + h_off)
    scode = jnp.pad(scode, ((0, 0), (0, SROW - H))).reshape(-1)
    feats_flat = features.reshape(-1)
    out = _sc_gather(feats_flat, scode)
    return out.reshape(B, P, C, H, 1)


# speedup vs baseline: 48.2624x; 4.7552x over previous
import functools

import jax
import jax.numpy as jnp
from jax import lax
from jax.experimental import pallas as pl
from jax.experimental.pallas import tpu as pltpu
from jax.experimental.pallas import tpu_sc as plsc

B = 16
C = 64
H = 11
W = 20
P = 2784
HW = H * W
NC = 2
NS = 16
NW = NC * NS
LANES = 16
NCH = P // LANES
FLOC = 2 * HW
SENTINEL = 1 << 20
RW = H * P


def _sc_body(feats_hbm, scode_hbm, out_hbm, scode_v, fbuf, obuf, sem0, sem1):
    cid = lax.axis_index("c")
    sid = lax.axis_index("s")
    wid = sid * NC + cid
    c0 = wid * 2

    pltpu.sync_copy(scode_hbm, scode_v)
    fbuf[pl.ds(FLOC, LANES)] = jnp.zeros((LANES,), jnp.float32)
    sems = (sem0, sem1)

    def batch_body(b, _):
        pltpu.sync_copy(feats_hbm.at[pl.ds((b * C + c0) * HW, FLOC)],
                        fbuf.at[pl.ds(0, FLOC)])
        for ci in range(2):
            out_slice = out_hbm.at[pl.ds((b * C + c0 + ci) * H * P, RW)]
            obuf_slice = obuf.at[pl.ds(ci * RW, RW)]

            @pl.when(b > 0)
            def _drain():
                pltpu.make_async_copy(out_slice, obuf_slice, sems[ci]).wait()

            for h in range(H):
                base = ci * HW + h * W
                row0 = (ci * H + h) * P

                @plsc.parallel_loop(0, NCH, 1, unroll=4)
                def chunk_body(i):
                    off = i * LANES
                    xvec = scode_v[pl.ds(h * P + off, LANES)]
                    iv = jnp.minimum(xvec + base, FLOC)
                    obuf[pl.ds(row0 + off, LANES)] = plsc.load_gather(
                        fbuf, [iv])

            pltpu.make_async_copy(obuf_slice, out_slice, sems[ci]).start()
        return 0

    lax.fori_loop(0, B, batch_body, 0)
    for ci in range(2):
        out_slice = out_hbm.at[pl.ds(((B - 1) * C + c0 + ci) * H * P, RW)]
        pltpu.make_async_copy(out_slice, obuf.at[pl.ds(ci * RW, RW)],
                              sems[ci]).wait()


_sc_gather = functools.partial(
    pl.kernel,
    out_type=jax.ShapeDtypeStruct((B * C * H * P,), jnp.float32),
    mesh=plsc.VectorSubcoreMesh(core_axis_name="c", subcore_axis_name="s",
                                num_cores=NC, num_subcores=NS),
    compiler_params=pltpu.CompilerParams(needs_layout_passes=False,
                                         disable_bounds_checks=True),
    scratch_types=[
        pltpu.VMEM((H * P,), jnp.int32),
        pltpu.VMEM((FLOC + LANES,), jnp.float32),
        pltpu.VMEM((2 * RW,), jnp.float32),
        pltpu.SemaphoreType.DMA,
        pltpu.SemaphoreType.DMA,
    ],
)(_sc_body)


def kernel(features, cut_zs, cut_ys, cut_xs, invalid_mask):
    del cut_zs, cut_ys
    xs = cut_xs.reshape(P, C, H)[:, 0, :]
    inv = invalid_mask.reshape(P, C, H)[:, 0, :]
    scode_t = jnp.where(inv, jnp.int32(SENTINEL), xs).T
    feats_flat = features.reshape(-1)
    out = _sc_gather(feats_flat, scode_t.reshape(-1))
    out = out.reshape(B, C, H, P).transpose(0, 3, 1, 2)
    return out[..., None]


# speedup vs baseline: 131.2033x; 2.7185x over previous
import functools

import jax
import jax.numpy as jnp
from jax import lax
from jax.experimental import pallas as pl
from jax.experimental.pallas import tpu as pltpu
from jax.experimental.pallas import tpu_sc as plsc

B = 16
C = 64
H = 11
W = 20
P = 2784
HW = H * W
NC = 2
NS = 16
NW = NC * NS
LANES = 16
NCH = P // LANES
FLOC = 2 * HW
SENTINEL = 1 << 20


def _sc_body(feats_hbm, scode_hbm, out_hbm, scode_v, fbuf, obuf, sem0, sem1):
    cid = lax.axis_index("c")
    sid = lax.axis_index("s")
    wid = sid * NC + cid
    c0 = wid * 2

    pltpu.sync_copy(scode_hbm, scode_v)
    fbuf[pl.ds(FLOC, LANES)] = jnp.zeros((LANES,), jnp.float32)
    sems = (sem0, sem1)

    def batch_body(b, _):
        pltpu.sync_copy(feats_hbm.at[pl.ds((b * C + c0) * HW, FLOC)],
                        fbuf.at[pl.ds(0, FLOC)])
        for ci in range(2):
            out_slice = out_hbm.at[b, c0 + ci]
            obuf_slice = obuf.at[ci]

            @pl.when(b > 0)
            def _drain():
                pltpu.make_async_copy(out_slice, obuf_slice, sems[ci]).wait()

            for h in range(H):
                base = ci * HW + h * W

                @plsc.parallel_loop(0, NCH, 1, unroll=4)
                def chunk_body(i):
                    off = i * LANES
                    xvec = scode_v[pl.ds(h * P + off, LANES)]
                    iv = jnp.minimum(xvec + base, FLOC)
                    obuf[ci, h, pl.ds(off, LANES)] = plsc.load_gather(
                        fbuf, [iv])

            pltpu.make_async_copy(obuf_slice, out_slice, sems[ci]).start()
        return 0

    lax.fori_loop(0, B, batch_body, 0)
    for ci in range(2):
        pltpu.make_async_copy(out_hbm.at[B - 1, c0 + ci], obuf.at[ci],
                              sems[ci]).wait()


_sc_gather = functools.partial(
    pl.kernel,
    out_type=jax.ShapeDtypeStruct((B, C, H, P), jnp.float32),
    mesh=plsc.VectorSubcoreMesh(core_axis_name="c", subcore_axis_name="s",
                                num_cores=NC, num_subcores=NS),
    compiler_params=pltpu.CompilerParams(needs_layout_passes=False,
                                         disable_bounds_checks=True),
    scratch_types=[
        pltpu.VMEM((H * P,), jnp.int32),
        pltpu.VMEM((FLOC + LANES,), jnp.float32),
        pltpu.VMEM((2, H, P), jnp.float32),
        pltpu.SemaphoreType.DMA,
        pltpu.SemaphoreType.DMA,
    ],
)(_sc_body)


def kernel(features, cut_zs, cut_ys, cut_xs, invalid_mask):
    del cut_zs, cut_ys
    pidx = jnp.arange(P, dtype=jnp.int32)[:, None]
    hidx = jnp.arange(H, dtype=jnp.int32)[None, :]
    xs = cut_xs[pidx * (C * H) + hidx]
    inv = invalid_mask[pidx, 0, hidx, 0]
    scode_t = jnp.where(inv, jnp.int32(SENTINEL), xs).T
    feats_flat = features.reshape(-1)
    out = _sc_gather(feats_flat, scode_t.reshape(-1))
    return out.transpose(0, 3, 1, 2)[..., None]
